# Initial kernel scaffold; baseline (speedup 1.0000x reference)
import jax
import jax.numpy as jnp
from jax.experimental import pallas as pl


def kernel(h, p, e, snorm_n, edge_index, W_pre_h, b_pre_h, W_pre_p, b_pre_p, W_post_h, b_post_h, W_post_p, b_post_p, bn_gamma, bn_beta, bn_mean, bn_var):
    raise NotImplementedError("write your pallas kernel here")


# speedup vs baseline: 1.8300x; 1.8300x over previous
import functools

import jax
import jax.numpy as jnp
from jax import lax
from jax.experimental import pallas as pl
from jax.experimental.pallas import tpu as pltpu
from jax.experimental.pallas import tpu_sc as plsc

N = 10000
E = 320000
D = 128
ED = 16
EPS = 1e-5

NC = 2
NS = 16
NW = NC * NS
RPW = 2
NR = NW * RPW
NPT = 160
NPAD = NR * NPT
TRASH = NPT

K = 1600
NCHUNK = E // K
G = 48
LISTCAP = K + 2 * G + 16
BIG = 3.0e38


def _agg_body(src_hbm, dst_hbm, a_hbm, c_hbm,
              s1_hbm, s2_hbm, mx_hbm, mn_hbm, dg_hbm,
              src_ch, dst_ch, slot_pend, src_pend, eid_pend,
              gidx_a, gidx_c, abuf, cbuf,
              acc_s1, acc_s2, acc_mx, acc_mn, acc_dg,
              sem_a, sem_c):
  wid = lax.axis_index("s") * NC + lax.axis_index("c")

  zeros_f = jnp.zeros((16,), jnp.float32)
  ones_f = jnp.full((16,), 1.0, jnp.float32)
  neg_big = jnp.full((16,), -BIG, jnp.float32)
  pos_big = jnp.full((16,), BIG, jnp.float32)
  zeros_i = jnp.zeros((16,), jnp.int32)
  trash_i = jnp.full((16,), TRASH, jnp.int32)

  def process_group(off):
    for q in range(G // 16):
      gidx_a[pl.ds(q * 16, 16)] = src_pend[pl.ds(off + q * 16, 16)]
      gidx_c[pl.ds(q * 16, 16)] = eid_pend[pl.ds(off + q * 16, 16)]
    cp_a = pltpu.make_async_copy(a_hbm.at[gidx_a], abuf, sem_a)
    cp_c = pltpu.make_async_copy(c_hbm.at[gidx_c], cbuf, sem_c)
    cp_a.start()
    cp_c.start()
    cp_a.wait()
    cp_c.wait()

    def sub(k, carry):
      slotv = slot_pend[pl.ds(off + k * 16, 16)]
      for l in range(16):
        s = slotv[l]
        row = k * 16 + l
        for j in range(D // 16):
          av = abuf[row, pl.ds(j * 16, 16)] + cbuf[row, pl.ds(j * 16, 16)]
          acc_s1[s, pl.ds(j * 16, 16)] += av
          acc_s2[s, pl.ds(j * 16, 16)] += av * av
          acc_mx[s, pl.ds(j * 16, 16)] = jnp.maximum(
              acc_mx[s, pl.ds(j * 16, 16)], av)
          acc_mn[s, pl.ds(j * 16, 16)] = jnp.minimum(
              acc_mn[s, pl.ds(j * 16, 16)], av)
        acc_dg[s, :] += ones_f
      return carry

    lax.fori_loop(0, G // 16, sub, 0)

  def do_range(rr, carry0):
    r = wid * RPW + rr
    lo = r * NPT

    def initrow(i, carry):
      for j in range(D // 16):
        acc_s1[i, pl.ds(j * 16, 16)] = zeros_f
        acc_s2[i, pl.ds(j * 16, 16)] = zeros_f
        acc_mx[i, pl.ds(j * 16, 16)] = neg_big
        acc_mn[i, pl.ds(j * 16, 16)] = pos_big
      acc_dg[i, :] = zeros_f
      return carry

    lax.fori_loop(0, NPT + 1, initrow, 0)

    def chunk_body(c, p_in):
      def scan_chunk(p0):
        cp1 = pltpu.make_async_copy(
            src_hbm.at[pl.ds(c * K, K)], src_ch, sem_a)
        cp2 = pltpu.make_async_copy(
            dst_hbm.at[pl.ds(c * K, K)], dst_ch, sem_c)
        cp1.start()
        cp2.start()
        cp1.wait()
        cp2.wait()

        def step(i, p2):
          sv = src_ch[pl.ds(i * 16, 16)]
          dv = dst_ch[pl.ds(i * 16, 16)]
          msk = (dv >= lo) & (dv < lo + NPT)
          slotv = dv - lo
          eidv = c * K + i * 16 + lax.iota(jnp.int32, 16)
          plsc.store_compressed(slot_pend.at[pl.ds(p2, 16)], slotv, mask=msk)
          plsc.store_compressed(src_pend.at[pl.ds(p2, 16)], sv, mask=msk)
          plsc.store_compressed(eid_pend.at[pl.ds(p2, 16)], eidv, mask=msk)
          pc = plsc.all_reduce_population_count(msk)[0]
          return p2 + pc

        return lax.fori_loop(0, K // 16, step, p0)

      def pad_tail(p0):
        for q in range(G // 16):
          slot_pend[pl.ds(p0 + q * 16, 16)] = trash_i
          src_pend[pl.ds(p0 + q * 16, 16)] = zeros_i
          eid_pend[pl.ds(p0 + q * 16, 16)] = zeros_i
        return p0 + G

      p = lax.cond(c < NCHUNK, scan_chunk, pad_tail, p_in)
      ng = p // G

      def drain(g, carry):
        process_group(g * G)
        return carry

      lax.fori_loop(0, ng, drain, 0)
      rem = p - ng * G
      base = ng * G
      for q in range(G // 16):
        slot_pend[pl.ds(q * 16, 16)] = slot_pend[pl.ds(base + q * 16, 16)]
        src_pend[pl.ds(q * 16, 16)] = src_pend[pl.ds(base + q * 16, 16)]
        eid_pend[pl.ds(q * 16, 16)] = eid_pend[pl.ds(base + q * 16, 16)]
      return rem

    lax.fori_loop(0, NCHUNK + 1, chunk_body, 0)

    pltpu.sync_copy(acc_s1.at[pl.ds(0, NPT)], s1_hbm.at[pl.ds(lo, NPT)])
    pltpu.sync_copy(acc_s2.at[pl.ds(0, NPT)], s2_hbm.at[pl.ds(lo, NPT)])
    pltpu.sync_copy(acc_mx.at[pl.ds(0, NPT)], mx_hbm.at[pl.ds(lo, NPT)])
    pltpu.sync_copy(acc_mn.at[pl.ds(0, NPT)], mn_hbm.at[pl.ds(lo, NPT)])
    pltpu.sync_copy(acc_dg.at[pl.ds(0, NPT)], dg_hbm.at[pl.ds(lo, NPT)])
    return carry0

  lax.fori_loop(0, RPW, do_range, 0)


def _make_agg():
  mesh = plsc.VectorSubcoreMesh(
      core_axis_name="c", subcore_axis_name="s",
      num_cores=NC, num_subcores=NS)
  return pl.kernel(
      _agg_body,
      compiler_params=pltpu.CompilerParams(needs_layout_passes=False),
      out_type=[
          jax.ShapeDtypeStruct((NPAD, D), jnp.float32),
          jax.ShapeDtypeStruct((NPAD, D), jnp.float32),
          jax.ShapeDtypeStruct((NPAD, D), jnp.float32),
          jax.ShapeDtypeStruct((NPAD, D), jnp.float32),
          jax.ShapeDtypeStruct((NPAD, 16), jnp.float32),
      ],
      mesh=mesh,
      scratch_types=[
          pltpu.VMEM((K,), jnp.int32),
          pltpu.VMEM((K,), jnp.int32),
          pltpu.VMEM((LISTCAP,), jnp.int32),
          pltpu.VMEM((LISTCAP,), jnp.int32),
          pltpu.VMEM((LISTCAP,), jnp.int32),
          pltpu.VMEM((G,), jnp.int32),
          pltpu.VMEM((G,), jnp.int32),
          pltpu.VMEM((G, D), jnp.float32),
          pltpu.VMEM((G, D), jnp.float32),
          pltpu.VMEM((NPT + 1, D), jnp.float32),
          pltpu.VMEM((NPT + 1, D), jnp.float32),
          pltpu.VMEM((NPT + 1, D), jnp.float32),
          pltpu.VMEM((NPT + 1, D), jnp.float32),
          pltpu.VMEM((NPT + 1, 16), jnp.float32),
          pltpu.SemaphoreType.DMA,
          pltpu.SemaphoreType.DMA,
      ],
  )


NBLK = 400
EBLK = 2000


def _pre_node_body(h_ref, p_ref, w_ref, ah_ref, bh_ref):
  hv = h_ref[...]
  pv = p_ref[...]
  w = w_ref[...]
  f32 = jnp.float32
  ah_ref[...] = (jnp.dot(hv, w[0:D], preferred_element_type=f32)
                 + jnp.dot(pv, w[D:2 * D], preferred_element_type=f32))
  bh_ref[...] = (jnp.dot(hv, w[2 * D:3 * D], preferred_element_type=f32)
                 + jnp.dot(pv, w[3 * D:4 * D], preferred_element_type=f32))


def _pre_edge_body(e_ref, w_ref, b_ref, c_ref):
  c_ref[...] = jnp.dot(e_ref[...], w_ref[...],
                       preferred_element_type=jnp.float32) + b_ref[...]


def _post_body(h_ref, p_ref, bh_ref, s1_ref, s2_ref, mxr_ref, mnr_ref,
               dg_ref, sn_ref, w_ref, b_ref, g_ref, bt_ref, bm_ref, bv_ref,
               ho_ref, po_ref):
  f32 = jnp.float32
  deg = dg_ref[...][:, 0:1]
  s1 = s1_ref[...]
  s2 = s2_ref[...]
  b = bh_ref[...]
  safe = jnp.maximum(deg, 1.0)
  mean = (s1 + deg * b) / safe
  sq = (s2 + 2.0 * b * s1 + deg * b * b) / safe
  var = jnp.maximum(sq - mean * mean, 0.0)
  std = jnp.sqrt(var + EPS)
  pos = deg > 0.0
  mx = jnp.where(pos, mxr_ref[...] + b, 0.0)
  mn = jnp.where(pos, mnr_ref[...] + b, 0.0)
  w = w_ref[...]
  hv = h_ref[...]
  pv = p_ref[...]
  h3 = (jnp.dot(hv, w[0:D], preferred_element_type=f32)
        + jnp.dot(pv, w[D:2 * D], preferred_element_type=f32)
        + jnp.dot(mean, w[2 * D:3 * D], preferred_element_type=f32)
        + jnp.dot(mx, w[3 * D:4 * D], preferred_element_type=f32)
        + jnp.dot(mn, w[4 * D:5 * D], preferred_element_type=f32)
        + jnp.dot(std, w[5 * D:6 * D], preferred_element_type=f32)
        + b_ref[...])
  h3 = h3 * sn_ref[...]
  scale = g_ref[...] * jax.lax.rsqrt(bv_ref[...] + EPS)
  h3 = (h3 - bm_ref[...]) * scale + bt_ref[...]
  h3 = jnp.maximum(h3, 0.0)
  ho_ref[...] = hv + h3
  po_ref[...] = pv + jnp.tanh(h3)


def _full(rows, cols):
  return pl.BlockSpec((rows, cols), lambda i: (0, 0))


_pre_node = pl.pallas_call(
    _pre_node_body,
    grid=(N // NBLK,),
    in_specs=[
        pl.BlockSpec((NBLK, D), lambda i: (i, 0)),
        pl.BlockSpec((NBLK, D), lambda i: (i, 0)),
        _full(4 * D, D),
    ],
    out_specs=[
        pl.BlockSpec((NBLK, D), lambda i: (i, 0)),
        pl.BlockSpec((NBLK, D), lambda i: (i, 0)),
    ],
    out_shape=[
        jax.ShapeDtypeStruct((N, D), jnp.float32),
        jax.ShapeDtypeStruct((N, D), jnp.float32),
    ],
)

_pre_edge = pl.pallas_call(
    _pre_edge_body,
    grid=(E // EBLK,),
    in_specs=[
        pl.BlockSpec((EBLK, ED), lambda i: (i, 0)),
        _full(ED, D),
        _full(1, D),
    ],
    out_specs=pl.BlockSpec((EBLK, D), lambda i: (i, 0)),
    out_shape=jax.ShapeDtypeStruct((E, D), jnp.float32),
)

_post = pl.pallas_call(
    _post_body,
    grid=(N // NBLK,),
    in_specs=[
        pl.BlockSpec((NBLK, D), lambda i: (i, 0)),
        pl.BlockSpec((NBLK, D), lambda i: (i, 0)),
        pl.BlockSpec((NBLK, D), lambda i: (i, 0)),
        pl.BlockSpec((NBLK, D), lambda i: (i, 0)),
        pl.BlockSpec((NBLK, D), lambda i: (i, 0)),
        pl.BlockSpec((NBLK, D), lambda i: (i, 0)),
        pl.BlockSpec((NBLK, D), lambda i: (i, 0)),
        pl.BlockSpec((NBLK, 16), lambda i: (i, 0)),
        pl.BlockSpec((NBLK, 1), lambda i: (i, 0)),
        _full(6 * D, D),
        _full(1, D),
        _full(1, D), _full(1, D), _full(1, D), _full(1, D),
    ],
    out_specs=[
        pl.BlockSpec((NBLK, D), lambda i: (i, 0)),
        pl.BlockSpec((NBLK, D), lambda i: (i, 0)),
    ],
    out_shape=[
        jax.ShapeDtypeStruct((N, D), jnp.float32),
        jax.ShapeDtypeStruct((N, D), jnp.float32),
    ],
)


def kernel(h, p, e, snorm_n, edge_index, W_pre_h, b_pre_h, W_pre_p, b_pre_p,
           W_post_h, b_post_h, W_post_p, b_post_p, bn_gamma, bn_beta,
           bn_mean, bn_var):
  del W_pre_p, b_pre_p, W_post_p, b_post_p
  src = edge_index[0]
  dst = edge_index[1]
  ah, bh = _pre_node(h, p, W_pre_h[:4 * D])
  ch = _pre_edge(e, W_pre_h[4 * D:], b_pre_h.reshape(1, D))
  s1, s2, mx, mn, dg = _make_agg()(src, dst, ah, ch)
  row = lambda v: v.reshape(1, D)
  h_out, p_out = _post(h, p, bh, s1, s2, mx, mn, dg, snorm_n,
                       W_post_h, row(b_post_h), row(bn_gamma), row(bn_beta),
                       row(bn_mean), row(bn_var))
  return (h_out, p_out)
